# Initial kernel scaffold; baseline (speedup 1.0000x reference)
import jax
import jax.numpy as jnp
from jax.experimental import pallas as pl


def kernel(txt, vals):
    raise NotImplementedError("write your pallas kernel here")


# speedup vs baseline: 21.9338x; 21.9338x over previous
import functools

import jax
import jax.numpy as jnp
from jax import lax
from jax.experimental import pallas as pl
from jax.experimental.pallas import tpu as pltpu
from jax.experimental.pallas import tpu_sc as plsc

N_TYPES = 1000
WINDOW = 1
DIM = (1 + 2 * WINDOW) * N_TYPES
B, L = 1024, 20
ROW = L * DIM
LP = 32

NC, NS, NL = 2, 16, 16
NW = NC * NS
BPW = B // NW


def _bow_sc(txt_hbm, vals_hbm, zeros_hbm, out_hbm, txt_v, vals_v,
            buf0, buf1, sem0, sem1, insem):
    wid = lax.axis_index("s") * NC + lax.axis_index("c")
    base = wid * (BPW * LP)

    pltpu.sync_copy(txt_hbm.at[pl.ds(base, BPW * LP)], txt_v)
    pltpu.sync_copy(vals_hbm.at[pl.ds(base, BPW * LP)], vals_v)

    c0 = pltpu.async_copy(zeros_hbm, buf0, insem)
    c1 = pltpu.async_copy(zeros_hbm, buf1, sem1)
    c0.wait()
    c1.wait()

    bufs = (buf0, buf1)
    sems = (sem0, sem1)
    iota = lax.iota(jnp.int32, NL)
    fzero = jnp.zeros((NL,), jnp.float32)

    pend = [None, None]
    dirty = [None, None]

    for b in range(BPW):
        p = b % 2
        buf = bufs[p]

        if pend[p] is not None:
            pend[p].wait()
            for idx, msk in dirty[p]:
                plsc.store_scatter(buf, [idx], fzero, mask=msk)

        writes = []
        for half in range(2):
            n = iota + (NL * half)
            off = b * LP + NL * half
            tok = txt_v[pl.ds(off, NL)]
            val = vals_v[pl.ds(off, NL)]
            h = lax.rem(tok, N_TYPES)
            rowbase = n * DIM
            in_seq = n < L
            idx_l = jnp.where(n == 0, (L - 1) * DIM + h, rowbase - DIM + h)
            msk_l = in_seq
            idx_m = rowbase + N_TYPES + h
            msk_m = in_seq
            idx_r = rowbase + DIM + 2 * N_TYPES + h
            msk_r = n < (L - 1)
            for idx, msk in ((idx_l, msk_l), (idx_m, msk_m), (idx_r, msk_r)):
                plsc.store_scatter(buf, [idx], val, mask=msk)
                writes.append((idx, msk))

        dirty[p] = writes
        pend[p] = pltpu.async_copy(buf, out_hbm.at[wid * BPW + b], sems[p])

    pend[0].wait()
    pend[1].wait()


@jax.jit
def _bow(txt, vals):
    txt_p = jnp.pad(txt, ((0, 0), (0, LP - L))).reshape(-1)
    vals_p = jnp.pad(vals, ((0, 0), (0, LP - L))).reshape(-1)
    zrow = jnp.zeros((ROW,), jnp.float32)
    mesh = plsc.VectorSubcoreMesh(core_axis_name="c", subcore_axis_name="s")
    out = pl.kernel(
        _bow_sc,
        mesh=mesh,
        compiler_params=pltpu.CompilerParams(needs_layout_passes=False),
        out_type=jax.ShapeDtypeStruct((B, ROW), jnp.float32),
        scratch_types=[
            pltpu.VMEM((BPW * LP,), jnp.int32),
            pltpu.VMEM((BPW * LP,), jnp.float32),
            pltpu.VMEM((ROW,), jnp.float32),
            pltpu.VMEM((ROW,), jnp.float32),
            pltpu.SemaphoreType.DMA,
            pltpu.SemaphoreType.DMA,
            pltpu.SemaphoreType.DMA,
        ],
    )(txt_p, vals_p, zrow)
    return out.reshape(B, L, DIM)


def kernel(txt, vals):
    return _bow(txt, vals)
